# baseline (speedup 1.0000x reference)
import functools

import jax
import jax.numpy as jnp
from jax import lax
from jax.experimental import pallas as pl
from jax.experimental.pallas import tpu as pltpu
from jax.experimental.pallas import tpu_sc as plsc

N = 2048
E = 16384
G = 16
EPS = 1e-6
F32 = jnp.float32


def _dot(a, b, dims):
    return lax.dot_general(a, b, (dims, ((), ())), preferred_element_type=F32)


def _fold_s(S):
    m1 = jnp.max(S[0:1, :]) + jnp.max(S[2:3, :])
    m2 = jnp.max(S[1:2, :]) + jnp.max(S[3:4, :])
    return jnp.concatenate(
        [S[0:1] - m1, S[1:2] - m2, S[2:4], jnp.zeros((4, N), F32)], axis=0)


def _proj0_body(x_ref, wu_ref, bu_ref, wv_ref, ws_ref, u_out, v_out, s_out):
    x = x_ref[...]
    u_out[...] = _dot(wu_ref[...], x, ((1,), (1,))) + bu_ref[...][:, None]
    v_out[...] = _dot(wv_ref[...], x, ((1,), (1,)))
    s_out[...] = _fold_s(_dot(ws_ref[...], x, ((1,), (1,))))


def _proj_mid_body(num_ref, den_ref, wu_ref, bu_ref, wv_ref, ws_ref,
                   u_out, v_out, s_out):
    num = num_ref[...]
    den = den_ref[...]
    half = num.shape[0] // 2
    hc = jnp.concatenate([num[:half] / (den[0:1] + EPS),
                          num[half:] / (den[1:2] + EPS)], axis=0)
    u_out[...] = _dot(wu_ref[...], hc, ((1,), (0,))) + bu_ref[...][:, None]
    v_out[...] = _dot(wv_ref[...], hc, ((1,), (0,)))
    s_out[...] = _fold_s(_dot(ws_ref[...], hc, ((1,), (0,))))


def _final_body(num_ref, den_ref, mg_ref, w1_ref, b1_ref, w2_ref, b2_ref,
                out_ref):
    num = num_ref[...]
    den = den_ref[...]
    half = num.shape[0] // 2
    hc = jnp.concatenate([num[:half] / (den[0:1] + EPS),
                          num[half:] / (den[1:2] + EPS)], axis=0)
    pooled = _dot(mg_ref[...], hc, ((0,), (1,)))
    z1 = jax.nn.relu(_dot(pooled, w1_ref[...], ((1,), (1,)))
                     + b1_ref[...][None, :])
    out_ref[...] = _dot(z1, w2_ref[...], ((1,), (1,))) + b2_ref[...][None, :]


def _tc_call(body, out_shapes, args):
    return pl.pallas_call(
        body,
        out_shape=[jax.ShapeDtypeStruct(s, F32) for s in out_shapes],
    )(*args)


@functools.cache
def _make_sc_edge(C, CPW):
    info = plsc.get_sparse_core_info()
    NC, NS = info.num_cores, info.num_subcores
    NW = NC * NS
    assert C == CPW * NW
    mesh = plsc.VectorSubcoreMesh(core_axis_name="c", subcore_axis_name="s")

    @functools.partial(
        pl.kernel, mesh=mesh,
        compiler_params=pltpu.CompilerParams(needs_layout_passes=False),
        out_type=[jax.ShapeDtypeStruct((C * N,), F32),
                  jax.ShapeDtypeStruct((2 * N,), F32)],
        scratch_types=[
            pltpu.VMEM((CPW * N,), F32),
            pltpu.VMEM((CPW * N,), F32),
            pltpu.VMEM((N,), F32),
            pltpu.VMEM((N,), F32),
            pltpu.VMEM((CPW * N,), F32),
            pltpu.VMEM((N,), F32),
            pltpu.VMEM((E,), jnp.int32),
            pltpu.VMEM((E,), jnp.int32),
        ],
    )
    def sc_edge(u_hbm, v_hbm, s_hbm, src_hbm, tgt_hbm, num_out, den_out,
                u_v, v_v, p_v, q_v, num_v, den_v, src_v, tgt_v):
        wid = lax.axis_index("s") * NC + lax.axis_index("c")
        head = wid // (NW // 2)
        r0 = pl.multiple_of(wid * (CPW * N), CPW * N)

        pltpu.sync_copy(u_hbm.at[pl.ds(r0, CPW * N)], u_v)
        pltpu.sync_copy(v_hbm.at[pl.ds(r0, CPW * N)], v_v)
        pltpu.sync_copy(s_hbm.at[pl.ds(pl.multiple_of(head * N, N), N)], p_v)
        pltpu.sync_copy(
            s_hbm.at[pl.ds(pl.multiple_of((2 + head) * N, N), N)], q_v)
        pltpu.sync_copy(src_hbm, src_v)
        pltpu.sync_copy(tgt_hbm, tgt_v)

        zf = jnp.zeros((16,), F32)

        @plsc.parallel_loop(0, CPW * N // 16, 1, unroll=8)
        def zero_num(j):
            num_v[pl.ds(pl.multiple_of(j * 16, 16), 16)] = zf

        @plsc.parallel_loop(0, N // 16, 1, unroll=8)
        def zero_den(j):
            den_v[pl.ds(pl.multiple_of(j * 16, 16), 16)] = zf

        @plsc.parallel_loop(0, E // 16, 1, unroll=8)
        def edge_body(g):
            base = pl.multiple_of(g * 16, 16)
            s16 = src_v[pl.ds(base, 16)]
            t16 = tgt_v[pl.ds(base, 16)]
            ps = plsc.load_gather(p_v, [s16])
            qt = plsc.load_gather(q_v, [t16])
            w = jnp.exp(ps + qt)
            plsc.addupdate_scatter(den_v, [t16], w)
            for c in range(CPW):
                us = plsc.load_gather(u_v, [s16 + (c * N)])
                vt = plsc.load_gather(v_v, [t16 + (c * N)])
                y = jnp.maximum(us + vt, 0.0)
                plsc.addupdate_scatter(num_v, [t16 + (c * N)], y * w)

        pltpu.sync_copy(num_v, num_out.at[pl.ds(r0, CPW * N)])

        @pl.when(jnp.logical_or(wid == 0, wid == NW // 2))
        def _():
            pltpu.sync_copy(
                den_v, den_out.at[pl.ds(pl.multiple_of(head * N, N), N)])

    return sc_edge


def _layer_weights(layer, d_in):
    wu = jnp.concatenate([hp["f"]["W"][:, :d_in] for hp in layer], axis=0)
    bu = jnp.concatenate([hp["f"]["b"] for hp in layer], axis=0)
    wv = jnp.concatenate([hp["f"]["W"][:, d_in:] for hp in layer], axis=0)
    ws = jnp.concatenate(
        [hp["w"]["W"][:, :d_in] for hp in layer]
        + [hp["w"]["W"][:, d_in:] for hp in layer], axis=0)
    return wu, bu, wv, ws


def _run_sc(c, u, v, s, src, tgt):
    num, den = _make_sc_edge(c, c // 32)(
        u.reshape(c * N), v.reshape(c * N), s[:4].reshape(4 * N), src, tgt)
    return num.reshape(c, N), den.reshape(2, N)


def kernel(x, adj, src, tgt, Msrc, Mtgt, Mgraph, params):
    del adj, Msrc, Mtgt
    gat = params["gat"]
    dims = [(128, 32), (64, 64), (128, 64)]

    wu, bu, wv, ws = _layer_weights(gat[0], dims[0][0])
    c1 = 2 * dims[0][1]
    u, v, s = _tc_call(_proj0_body, [(c1, N), (c1, N), (8, N)],
                       (x, wu, bu, wv, ws))
    num, den = _run_sc(c1, u, v, s, src, tgt)

    for li in (1, 2):
        wu, bu, wv, ws = _layer_weights(gat[li], dims[li][0])
        cl = 2 * dims[li][1]
        u, v, s = _tc_call(_proj_mid_body, [(cl, N), (cl, N), (8, N)],
                           (num, den, wu, bu, wv, ws))
        num, den = _run_sc(cl, u, v, s, src, tgt)

    (out,) = _tc_call(
        _final_body, [(G, 10)],
        (num, den, Mgraph,
         params["mlp"][0]["W"], params["mlp"][0]["b"],
         params["mlp"][1]["W"], params["mlp"][1]["b"]))
    return out


# speedup vs baseline: 1.0399x; 1.0399x over previous
import functools

import jax
import jax.numpy as jnp
from jax import lax
from jax.experimental import pallas as pl
from jax.experimental.pallas import tpu as pltpu
from jax.experimental.pallas import tpu_sc as plsc

N = 2048
E = 16384
G = 16
EPS = 1e-6
F32 = jnp.float32


def _dot(a, b, dims):
    return lax.dot_general(a, b, (dims, ((), ())), preferred_element_type=F32)


def _fold_s(S):
    m1 = jnp.max(S[0:1, :]) + jnp.max(S[2:3, :])
    m2 = jnp.max(S[1:2, :]) + jnp.max(S[3:4, :])
    return jnp.concatenate(
        [S[0:1] - m1, S[1:2] - m2, S[2:4], jnp.zeros((4, N), F32)], axis=0)


def _proj0_body(x_ref, wu_ref, bu_ref, wv_ref, ws_ref, u_out, v_out, s_out):
    x = x_ref[...]
    u_out[...] = _dot(wu_ref[...], x, ((1,), (1,))) + bu_ref[...][:, None]
    v_out[...] = _dot(wv_ref[...], x, ((1,), (1,)))
    s_out[...] = _fold_s(_dot(ws_ref[...], x, ((1,), (1,))))


def _proj_mid_body(num_ref, den_ref, wu_ref, bu_ref, wv_ref, ws_ref,
                   u_out, v_out, s_out):
    num = num_ref[0] + num_ref[1]
    den = den_ref[0] + den_ref[1]
    half = num.shape[0] // 2
    hc = jnp.concatenate([num[:half] / (den[0:1] + EPS),
                          num[half:] / (den[1:2] + EPS)], axis=0)
    u_out[...] = _dot(wu_ref[...], hc, ((1,), (0,))) + bu_ref[...][:, None]
    v_out[...] = _dot(wv_ref[...], hc, ((1,), (0,)))
    s_out[...] = _fold_s(_dot(ws_ref[...], hc, ((1,), (0,))))


def _final_body(num_ref, den_ref, mg_ref, w1_ref, b1_ref, w2_ref, b2_ref,
                out_ref):
    num = num_ref[0] + num_ref[1]
    den = den_ref[0] + den_ref[1]
    half = num.shape[0] // 2
    hc = jnp.concatenate([num[:half] / (den[0:1] + EPS),
                          num[half:] / (den[1:2] + EPS)], axis=0)
    pooled = _dot(mg_ref[...], hc, ((0,), (1,)))
    z1 = jax.nn.relu(_dot(pooled, w1_ref[...], ((1,), (1,)))
                     + b1_ref[...][None, :])
    out_ref[...] = _dot(z1, w2_ref[...], ((1,), (1,))) + b2_ref[...][None, :]


def _tc_call(body, out_shapes, args):
    return pl.pallas_call(
        body,
        out_shape=[jax.ShapeDtypeStruct(s, F32) for s in out_shapes],
    )(*args)


@functools.cache
def _make_sc_edge(C, CPW):
    info = plsc.get_sparse_core_info()
    NC, NS = info.num_cores, info.num_subcores
    assert C == CPW * NS
    EH = E // NC
    mesh = plsc.VectorSubcoreMesh(core_axis_name="c", subcore_axis_name="s")

    @functools.partial(
        pl.kernel, mesh=mesh,
        compiler_params=pltpu.CompilerParams(needs_layout_passes=False),
        out_type=[jax.ShapeDtypeStruct((NC * C * N,), F32),
                  jax.ShapeDtypeStruct((NC * 2 * N,), F32)],
        scratch_types=[
            pltpu.VMEM((CPW * N,), F32),
            pltpu.VMEM((CPW * N,), F32),
            pltpu.VMEM((N,), F32),
            pltpu.VMEM((N,), F32),
            pltpu.VMEM((CPW * N,), F32),
            pltpu.VMEM((N,), F32),
            pltpu.VMEM((EH,), jnp.int32),
            pltpu.VMEM((EH,), jnp.int32),
        ],
    )
    def sc_edge(u_hbm, v_hbm, s_hbm, src_hbm, tgt_hbm, num_out, den_out,
                u_v, v_v, p_v, q_v, num_v, den_v, src_v, tgt_v):
        sc = lax.axis_index("c")
        sub = lax.axis_index("s")
        head = sub // (NS // 2)
        r0 = pl.multiple_of(sub * (CPW * N), CPW * N)
        e0 = pl.multiple_of(sc * EH, EH)

        pltpu.sync_copy(u_hbm.at[pl.ds(r0, CPW * N)], u_v)
        pltpu.sync_copy(v_hbm.at[pl.ds(r0, CPW * N)], v_v)
        pltpu.sync_copy(s_hbm.at[pl.ds(pl.multiple_of(head * N, N), N)], p_v)
        pltpu.sync_copy(
            s_hbm.at[pl.ds(pl.multiple_of((2 + head) * N, N), N)], q_v)
        pltpu.sync_copy(src_hbm.at[pl.ds(e0, EH)], src_v)
        pltpu.sync_copy(tgt_hbm.at[pl.ds(e0, EH)], tgt_v)

        zf = jnp.zeros((16,), F32)

        @plsc.parallel_loop(0, CPW * N // 16, 1, unroll=8)
        def zero_num(j):
            num_v[pl.ds(pl.multiple_of(j * 16, 16), 16)] = zf

        @plsc.parallel_loop(0, N // 16, 1, unroll=8)
        def zero_den(j):
            den_v[pl.ds(pl.multiple_of(j * 16, 16), 16)] = zf

        @plsc.parallel_loop(0, EH // 16, 1, unroll=4)
        def edge_body(g):
            base = pl.multiple_of(g * 16, 16)
            s16 = src_v[pl.ds(base, 16)]
            t16 = tgt_v[pl.ds(base, 16)]
            ps = plsc.load_gather(p_v, [s16])
            qt = plsc.load_gather(q_v, [t16])
            w = jnp.exp(ps + qt)
            plsc.addupdate_scatter(den_v, [t16], w)
            for c in range(CPW):
                us = plsc.load_gather(u_v, [s16 + (c * N)])
                vt = plsc.load_gather(v_v, [t16 + (c * N)])
                y = jnp.maximum(us + vt, 0.0)
                plsc.addupdate_scatter(num_v, [t16 + (c * N)], y * w)

        pltpu.sync_copy(
            num_v, num_out.at[pl.ds(pl.multiple_of(sc * (C * N), C * N)
                                    + r0, CPW * N)])

        @pl.when(jnp.logical_or(sub == 0, sub == NS // 2))
        def _():
            pltpu.sync_copy(
                den_v,
                den_out.at[pl.ds(pl.multiple_of(sc * (2 * N), 2 * N)
                                 + pl.multiple_of(head * N, N), N)])

    return sc_edge


def _layer_weights(layer, d_in):
    wu = jnp.concatenate([hp["f"]["W"][:, :d_in] for hp in layer], axis=0)
    bu = jnp.concatenate([hp["f"]["b"] for hp in layer], axis=0)
    wv = jnp.concatenate([hp["f"]["W"][:, d_in:] for hp in layer], axis=0)
    ws = jnp.concatenate(
        [hp["w"]["W"][:, :d_in] for hp in layer]
        + [hp["w"]["W"][:, d_in:] for hp in layer], axis=0)
    return wu, bu, wv, ws


def _run_sc(c, u, v, s, src, tgt):
    num, den = _make_sc_edge(c, c // 16)(
        u.reshape(c * N), v.reshape(c * N), s[:4].reshape(4 * N), src, tgt)
    return num.reshape(2, c, N), den.reshape(2, 2, N)


def kernel(x, adj, src, tgt, Msrc, Mtgt, Mgraph, params):
    del adj, Msrc, Mtgt
    gat = params["gat"]
    dims = [(128, 32), (64, 64), (128, 64)]

    wu, bu, wv, ws = _layer_weights(gat[0], dims[0][0])
    c1 = 2 * dims[0][1]
    u, v, s = _tc_call(_proj0_body, [(c1, N), (c1, N), (8, N)],
                       (x, wu, bu, wv, ws))
    num, den = _run_sc(c1, u, v, s, src, tgt)

    for li in (1, 2):
        wu, bu, wv, ws = _layer_weights(gat[li], dims[li][0])
        cl = 2 * dims[li][1]
        u, v, s = _tc_call(_proj_mid_body, [(cl, N), (cl, N), (8, N)],
                           (num, den, wu, bu, wv, ws))
        num, den = _run_sc(cl, u, v, s, src, tgt)

    (out,) = _tc_call(
        _final_body, [(G, 10)],
        (num, den, Mgraph,
         params["mlp"][0]["W"], params["mlp"][0]["b"],
         params["mlp"][1]["W"], params["mlp"][1]["b"]))
    return out


# speedup vs baseline: 1.2759x; 1.2270x over previous
import functools

import jax
import jax.numpy as jnp
from jax import lax
from jax.experimental import pallas as pl
from jax.experimental.pallas import tpu as pltpu
from jax.experimental.pallas import tpu_sc as plsc

N = 2048
E = 16384
G = 16
EPS = 1e-6
F32 = jnp.float32


def _dot(a, b, dims):
    return lax.dot_general(a, b, (dims, ((), ())), preferred_element_type=F32)


def _fold_s(S):
    m1 = jnp.max(S[0:1, :]) + jnp.max(S[2:3, :])
    m2 = jnp.max(S[1:2, :]) + jnp.max(S[3:4, :])
    return jnp.concatenate(
        [S[0:1] - m1, S[1:2] - m2, S[2:4], jnp.zeros((4, N), F32)], axis=0)


def _proj0_body(x_ref, wu_ref, bu_ref, wv_ref, ws_ref, u_out, v_out, s_out):
    x = x_ref[...]
    u_out[...] = _dot(wu_ref[...], x, ((1,), (1,))) + bu_ref[...][:, None]
    v_out[...] = _dot(wv_ref[...], x, ((1,), (1,)))
    s_out[...] = _fold_s(_dot(ws_ref[...], x, ((1,), (1,))))


def _proj_mid_body(num_ref, den_ref, wu_ref, bu_ref, wv_ref, ws_ref,
                   u_out, v_out, s_out):
    c_in = num_ref.shape[0] // 2
    num = num_ref[:c_in] + num_ref[c_in:]
    den = den_ref[:2] + den_ref[2:]
    half = c_in // 2
    hc = jnp.concatenate([num[:half] / (den[0:1] + EPS),
                          num[half:] / (den[1:2] + EPS)], axis=0)
    u_out[...] = _dot(wu_ref[...], hc, ((1,), (0,))) + bu_ref[...][:, None]
    v_out[...] = _dot(wv_ref[...], hc, ((1,), (0,)))
    s_out[...] = _fold_s(_dot(ws_ref[...], hc, ((1,), (0,))))


def _final_body(num_ref, den_ref, mg_ref, w1_ref, b1_ref, w2_ref, b2_ref,
                out_ref):
    c_in = num_ref.shape[0] // 2
    num = num_ref[:c_in] + num_ref[c_in:]
    den = den_ref[:2] + den_ref[2:]
    half = c_in // 2
    hc = jnp.concatenate([num[:half] / (den[0:1] + EPS),
                          num[half:] / (den[1:2] + EPS)], axis=0)
    pooled = _dot(mg_ref[...], hc, ((0,), (1,)))
    z1 = jax.nn.relu(_dot(pooled, w1_ref[...], ((1,), (1,)))
                     + b1_ref[...][None, :])
    out_ref[...] = _dot(z1, w2_ref[...], ((1,), (1,))) + b2_ref[...][None, :]


def _tc_call(body, out_shapes, args):
    return pl.pallas_call(
        body,
        out_shape=[jax.ShapeDtypeStruct(s, F32) for s in out_shapes],
    )(*args)


@functools.cache
def _make_sc_edge(C, CPW):
    info = plsc.get_sparse_core_info()
    NC, NS = info.num_cores, info.num_subcores
    assert C == CPW * NS
    EH = E // NC
    mesh = plsc.VectorSubcoreMesh(core_axis_name="c", subcore_axis_name="s")

    @functools.partial(
        pl.kernel, mesh=mesh,
        compiler_params=pltpu.CompilerParams(needs_layout_passes=False),
        out_type=[jax.ShapeDtypeStruct((NC * C, N), F32),
                  jax.ShapeDtypeStruct((NC * 2, N), F32)],
        scratch_types=[
            pltpu.VMEM((CPW, N), F32),
            pltpu.VMEM((CPW, N), F32),
            pltpu.VMEM((1, N), F32),
            pltpu.VMEM((1, N), F32),
            pltpu.VMEM((CPW, N), F32),
            pltpu.VMEM((1, N), F32),
            pltpu.VMEM((EH,), jnp.int32),
            pltpu.VMEM((EH,), jnp.int32),
        ],
    )
    def sc_edge(u_hbm, v_hbm, s_hbm, src_hbm, tgt_hbm, num_out, den_out,
                u_v, v_v, p_v, q_v, num_v, den_v, src_v, tgt_v):
        sc = lax.axis_index("c")
        sub = lax.axis_index("s")
        head = sub // (NS // 2)
        r0 = pl.multiple_of(sub * CPW, CPW)
        e0 = pl.multiple_of(sc * EH, EH)

        pltpu.sync_copy(u_hbm.at[pl.ds(r0, CPW)], u_v)
        pltpu.sync_copy(v_hbm.at[pl.ds(r0, CPW)], v_v)
        pltpu.sync_copy(s_hbm.at[pl.ds(head, 1)], p_v)
        pltpu.sync_copy(s_hbm.at[pl.ds(2 + head, 1)], q_v)
        pltpu.sync_copy(src_hbm.at[pl.ds(e0, EH)], src_v)
        pltpu.sync_copy(tgt_hbm.at[pl.ds(e0, EH)], tgt_v)

        zf = jnp.zeros((16,), F32)
        zi = jnp.zeros((16,), jnp.int32)

        @plsc.parallel_loop(0, N // 16, 1, unroll=8)
        def zero_acc(j):
            off = pl.multiple_of(j * 16, 16)
            for c in range(CPW):
                num_v[c, pl.ds(off, 16)] = zf
            den_v[0, pl.ds(off, 16)] = zf

        @plsc.parallel_loop(0, EH // 16, 1, unroll=4)
        def edge_body(g):
            base = pl.multiple_of(g * 16, 16)
            s16 = src_v[pl.ds(base, 16)]
            t16 = tgt_v[pl.ds(base, 16)]
            ps = plsc.load_gather(p_v, [zi, s16])
            qt = plsc.load_gather(q_v, [zi, t16])
            w = jnp.exp(ps + qt)
            plsc.addupdate_scatter(den_v, [zi, t16], w)
            for c in range(CPW):
                cv = jnp.full((16,), c, jnp.int32)
                us = plsc.load_gather(u_v, [cv, s16])
                vt = plsc.load_gather(v_v, [cv, t16])
                y = jnp.maximum(us + vt, 0.0)
                plsc.addupdate_scatter(num_v, [cv, t16], y * w)

        pltpu.sync_copy(num_v, num_out.at[pl.ds(sc * C + r0, CPW)])

        @pl.when(jnp.logical_or(sub == 0, sub == NS // 2))
        def _():
            pltpu.sync_copy(den_v, den_out.at[pl.ds(sc * 2 + head, 1)])

    return sc_edge


def _layer_weights(layer, d_in):
    wu = jnp.concatenate([hp["f"]["W"][:, :d_in] for hp in layer], axis=0)
    bu = jnp.concatenate([hp["f"]["b"] for hp in layer], axis=0)
    wv = jnp.concatenate([hp["f"]["W"][:, d_in:] for hp in layer], axis=0)
    ws = jnp.concatenate(
        [hp["w"]["W"][:, :d_in] for hp in layer]
        + [hp["w"]["W"][:, d_in:] for hp in layer], axis=0)
    return wu, bu, wv, ws


def _run_sc(c, u, v, s, src, tgt):
    return _make_sc_edge(c, c // 16)(u, v, s, src, tgt)


def kernel(x, adj, src, tgt, Msrc, Mtgt, Mgraph, params):
    del adj, Msrc, Mtgt
    gat = params["gat"]
    dims = [(128, 32), (64, 64), (128, 64)]

    wu, bu, wv, ws = _layer_weights(gat[0], dims[0][0])
    c1 = 2 * dims[0][1]
    u, v, s = _tc_call(_proj0_body, [(c1, N), (c1, N), (8, N)],
                       (x, wu, bu, wv, ws))
    num, den = _run_sc(c1, u, v, s, src, tgt)

    for li in (1, 2):
        wu, bu, wv, ws = _layer_weights(gat[li], dims[li][0])
        cl = 2 * dims[li][1]
        u, v, s = _tc_call(_proj_mid_body, [(cl, N), (cl, N), (8, N)],
                           (num, den, wu, bu, wv, ws))
        num, den = _run_sc(cl, u, v, s, src, tgt)

    (out,) = _tc_call(
        _final_body, [(G, 10)],
        (num, den, Mgraph,
         params["mlp"][0]["W"], params["mlp"][0]["b"],
         params["mlp"][1]["W"], params["mlp"][1]["b"]))
    return out


# speedup vs baseline: 1.3562x; 1.0629x over previous
import functools

import jax
import jax.numpy as jnp
from jax import lax
from jax.experimental import pallas as pl
from jax.experimental.pallas import tpu as pltpu
from jax.experimental.pallas import tpu_sc as plsc

N = 2048
E = 16384
G = 16
EPS = 1e-6
F32 = jnp.float32


def _dot(a, b, dims):
    return lax.dot_general(a, b, (dims, ((), ())), preferred_element_type=F32)


def _fold_s(S):
    m1 = jnp.max(S[0:1, :]) + jnp.max(S[2:3, :])
    m2 = jnp.max(S[1:2, :]) + jnp.max(S[3:4, :])
    return jnp.concatenate(
        [S[0:1] - m1, S[1:2] - m2, S[2:4], jnp.zeros((4, N), F32)], axis=0)


def _split_weights(d_in, w0f_ref, b0_ref, w1f_ref, b1_ref, w0w_ref, w1w_ref):
    w0, w1 = w0f_ref[...], w1f_ref[...]
    wu = jnp.concatenate([w0[:, :d_in], w1[:, :d_in]], axis=0)
    wv = jnp.concatenate([w0[:, d_in:], w1[:, d_in:]], axis=0)
    bu = jnp.concatenate([b0_ref[...], b1_ref[...]], axis=0)
    a0, a1 = w0w_ref[...], w1w_ref[...]
    ws = jnp.concatenate(
        [a0[:, :d_in], a1[:, :d_in], a0[:, d_in:], a1[:, d_in:]], axis=0)
    return wu, wv, bu, ws


def _proj0_body(d_in, x_ref, w0f_ref, b0_ref, w1f_ref, b1_ref, w0w_ref,
                w1w_ref, u_out, v_out, s_out):
    wu, wv, bu, ws = _split_weights(
        d_in, w0f_ref, b0_ref, w1f_ref, b1_ref, w0w_ref, w1w_ref)
    x = x_ref[...]
    u_out[...] = _dot(wu, x, ((1,), (1,))) + bu[:, None]
    v_out[...] = _dot(wv, x, ((1,), (1,)))
    s_out[...] = _fold_s(_dot(ws, x, ((1,), (1,))))


def _proj_mid_body(d_in, num_ref, den_ref, w0f_ref, b0_ref, w1f_ref, b1_ref,
                   w0w_ref, w1w_ref, u_out, v_out, s_out):
    wu, wv, bu, ws = _split_weights(
        d_in, w0f_ref, b0_ref, w1f_ref, b1_ref, w0w_ref, w1w_ref)
    c_in = num_ref.shape[0] // 2
    num = num_ref[:c_in] + num_ref[c_in:]
    den = den_ref[:2] + den_ref[2:]
    half = c_in // 2
    hc = jnp.concatenate([num[:half] / (den[0:1] + EPS),
                          num[half:] / (den[1:2] + EPS)], axis=0)
    u_out[...] = _dot(wu, hc, ((1,), (0,))) + bu[:, None]
    v_out[...] = _dot(wv, hc, ((1,), (0,)))
    s_out[...] = _fold_s(_dot(ws, hc, ((1,), (0,))))


def _final_body(num_ref, den_ref, mg_ref, w1_ref, b1_ref, w2_ref, b2_ref,
                out_ref):
    c_in = num_ref.shape[0] // 2
    num = num_ref[:c_in] + num_ref[c_in:]
    den = den_ref[:2] + den_ref[2:]
    half = c_in // 2
    hc = jnp.concatenate([num[:half] / (den[0:1] + EPS),
                          num[half:] / (den[1:2] + EPS)], axis=0)
    pooled = _dot(mg_ref[...], hc, ((0,), (1,)))
    z1 = jax.nn.relu(_dot(pooled, w1_ref[...], ((1,), (1,)))
                     + b1_ref[...][None, :])
    out_ref[...] = _dot(z1, w2_ref[...], ((1,), (1,))) + b2_ref[...][None, :]


def _tc_call(body, out_shapes, args):
    return pl.pallas_call(
        body,
        out_shape=[jax.ShapeDtypeStruct(s, F32) for s in out_shapes],
    )(*args)


@functools.cache
def _make_sc_edge(C, CPW):
    info = plsc.get_sparse_core_info()
    NC, NS = info.num_cores, info.num_subcores
    assert C == CPW * NS
    EH = E // NC
    mesh = plsc.VectorSubcoreMesh(core_axis_name="c", subcore_axis_name="s")

    @functools.partial(
        pl.kernel, mesh=mesh,
        compiler_params=pltpu.CompilerParams(needs_layout_passes=False),
        out_type=[jax.ShapeDtypeStruct((NC * C, N), F32),
                  jax.ShapeDtypeStruct((NC * 2, N), F32)],
        scratch_types=[
            pltpu.VMEM((CPW, N), F32),
            pltpu.VMEM((CPW, N), F32),
            pltpu.VMEM((1, N), F32),
            pltpu.VMEM((1, N), F32),
            pltpu.VMEM((CPW, N), F32),
            pltpu.VMEM((1, N), F32),
            pltpu.VMEM((EH,), jnp.int32),
            pltpu.VMEM((EH,), jnp.int32),
        ],
    )
    def sc_edge(u_hbm, v_hbm, s_hbm, src_hbm, tgt_hbm, num_out, den_out,
                u_v, v_v, p_v, q_v, num_v, den_v, src_v, tgt_v):
        sc = lax.axis_index("c")
        sub = lax.axis_index("s")
        head = sub // (NS // 2)
        r0 = pl.multiple_of(sub * CPW, CPW)
        e0 = pl.multiple_of(sc * EH, EH)

        pltpu.sync_copy(u_hbm.at[pl.ds(r0, CPW)], u_v)
        pltpu.sync_copy(v_hbm.at[pl.ds(r0, CPW)], v_v)
        pltpu.sync_copy(s_hbm.at[pl.ds(head, 1)], p_v)
        pltpu.sync_copy(s_hbm.at[pl.ds(2 + head, 1)], q_v)
        pltpu.sync_copy(src_hbm.at[pl.ds(e0, EH)], src_v)
        pltpu.sync_copy(tgt_hbm.at[pl.ds(e0, EH)], tgt_v)

        zf = jnp.zeros((16,), F32)
        zi = jnp.zeros((16,), jnp.int32)

        @plsc.parallel_loop(0, N // 16, 1, unroll=8)
        def zero_acc(j):
            off = pl.multiple_of(j * 16, 16)
            for c in range(CPW):
                num_v[c, pl.ds(off, 16)] = zf
            den_v[0, pl.ds(off, 16)] = zf

        @plsc.parallel_loop(0, EH // 16, 1, unroll=4)
        def edge_body(g):
            base = pl.multiple_of(g * 16, 16)
            s16 = src_v[pl.ds(base, 16)]
            t16 = tgt_v[pl.ds(base, 16)]
            ps = plsc.load_gather(p_v, [zi, s16])
            qt = plsc.load_gather(q_v, [zi, t16])
            w = jnp.exp(ps + qt)
            plsc.addupdate_scatter(den_v, [zi, t16], w)
            for c in range(CPW):
                cv = jnp.full((16,), c, jnp.int32)
                us = plsc.load_gather(u_v, [cv, s16])
                vt = plsc.load_gather(v_v, [cv, t16])
                y = jnp.maximum(us + vt, 0.0)
                plsc.addupdate_scatter(num_v, [cv, t16], y * w)

        pltpu.sync_copy(num_v, num_out.at[pl.ds(sc * C + r0, CPW)])

        @pl.when(jnp.logical_or(sub == 0, sub == NS // 2))
        def _():
            pltpu.sync_copy(den_v, den_out.at[pl.ds(sc * 2 + head, 1)])

    return sc_edge


def _run_sc(c, u, v, s, src, tgt):
    return _make_sc_edge(c, c // 16)(u, v, s, src, tgt)


def _wargs(layer):
    h0, h1 = layer
    return (h0["f"]["W"], h0["f"]["b"], h1["f"]["W"], h1["f"]["b"],
            h0["w"]["W"], h1["w"]["W"])


def kernel(x, adj, src, tgt, Msrc, Mtgt, Mgraph, params):
    del adj, Msrc, Mtgt
    gat = params["gat"]
    dims = [(128, 32), (64, 64), (128, 64)]

    c1 = 2 * dims[0][1]
    u, v, s = _tc_call(functools.partial(_proj0_body, dims[0][0]),
                       [(c1, N), (c1, N), (8, N)], (x, *_wargs(gat[0])))
    num, den = _run_sc(c1, u, v, s, src, tgt)

    for li in (1, 2):
        cl = 2 * dims[li][1]
        u, v, s = _tc_call(functools.partial(_proj_mid_body, dims[li][0]),
                           [(cl, N), (cl, N), (8, N)],
                           (num, den, *_wargs(gat[li])))
        num, den = _run_sc(cl, u, v, s, src, tgt)

    (out,) = _tc_call(
        _final_body, [(G, 10)],
        (num, den, Mgraph,
         params["mlp"][0]["W"], params["mlp"][0]["b"],
         params["mlp"][1]["W"], params["mlp"][1]["b"]))
    return out


# speedup vs baseline: 1.3805x; 1.0180x over previous
import functools

import jax
import jax.numpy as jnp
from jax import lax
from jax.experimental import pallas as pl
from jax.experimental.pallas import tpu as pltpu
from jax.experimental.pallas import tpu_sc as plsc

N = 2048
E = 16384
G = 16
EPS = 1e-6
F32 = jnp.float32


def _dot(a, b, dims):
    return lax.dot_general(a, b, (dims, ((), ())), preferred_element_type=F32)


def _fold_s(S):
    m1 = jnp.max(S[0:1, :]) + jnp.max(S[2:3, :])
    m2 = jnp.max(S[1:2, :]) + jnp.max(S[3:4, :])
    return jnp.concatenate(
        [S[0:1] - m1, S[1:2] - m2, S[2:4], jnp.zeros((4, N), F32)], axis=0)


def _split_weights(d_in, w0f_ref, b0_ref, w1f_ref, b1_ref, w0w_ref, w1w_ref):
    w0, w1 = w0f_ref[...], w1f_ref[...]
    wu = jnp.concatenate([w0[:, :d_in], w1[:, :d_in]], axis=0)
    wv = jnp.concatenate([w0[:, d_in:], w1[:, d_in:]], axis=0)
    bu = jnp.concatenate([b0_ref[...], b1_ref[...]], axis=0)
    a0, a1 = w0w_ref[...], w1w_ref[...]
    ws = jnp.concatenate(
        [a0[:, :d_in], a1[:, :d_in], a0[:, d_in:], a1[:, d_in:]], axis=0)
    return wu, wv, bu, ws


def _proj0_body(d_in, x_ref, w0f_ref, b0_ref, w1f_ref, b1_ref, w0w_ref,
                w1w_ref, u_out, v_out, s_out):
    wu, wv, bu, ws = _split_weights(
        d_in, w0f_ref, b0_ref, w1f_ref, b1_ref, w0w_ref, w1w_ref)
    x = x_ref[...]
    u_out[...] = _dot(wu, x, ((1,), (1,))) + bu[:, None]
    v_out[...] = _dot(wv, x, ((1,), (1,)))
    s_out[...] = _fold_s(_dot(ws, x, ((1,), (1,))))


def _proj_mid_body(d_in, num_ref, den_ref, w0f_ref, b0_ref, w1f_ref, b1_ref,
                   w0w_ref, w1w_ref, u_out, v_out, s_out):
    wu, wv, bu, ws = _split_weights(
        d_in, w0f_ref, b0_ref, w1f_ref, b1_ref, w0w_ref, w1w_ref)
    c_in = num_ref.shape[0] // 2
    num = num_ref[:c_in] + num_ref[c_in:]
    den = den_ref[:2] + den_ref[2:]
    half = c_in // 2
    hc = jnp.concatenate([num[:half] / (den[0:1] + EPS),
                          num[half:] / (den[1:2] + EPS)], axis=0)
    u_out[...] = _dot(wu, hc, ((1,), (0,))) + bu[:, None]
    v_out[...] = _dot(wv, hc, ((1,), (0,)))
    s_out[...] = _fold_s(_dot(ws, hc, ((1,), (0,))))


def _final_body(num_ref, den_ref, mg_ref, w1_ref, b1_ref, w2_ref, b2_ref,
                out_ref):
    c_in = num_ref.shape[0] // 2
    num = num_ref[:c_in] + num_ref[c_in:]
    den = den_ref[:2] + den_ref[2:]
    half = c_in // 2
    hc = jnp.concatenate([num[:half] / (den[0:1] + EPS),
                          num[half:] / (den[1:2] + EPS)], axis=0)
    pooled = _dot(mg_ref[...], hc, ((0,), (1,)))
    z1 = jax.nn.relu(_dot(pooled, w1_ref[...], ((1,), (1,)))
                     + b1_ref[...][None, :])
    out_ref[...] = _dot(z1, w2_ref[...], ((1,), (1,))) + b2_ref[...][None, :]


def _tc_call(body, out_shapes, args):
    return pl.pallas_call(
        body,
        out_shape=[jax.ShapeDtypeStruct(s, F32) for s in out_shapes],
    )(*args)


@functools.cache
def _make_sc_edge(C, CPW):
    info = plsc.get_sparse_core_info()
    NC, NS = info.num_cores, info.num_subcores
    assert C == CPW * NS
    EH = E // NC
    mesh = plsc.VectorSubcoreMesh(core_axis_name="c", subcore_axis_name="s")

    @functools.partial(
        pl.kernel, mesh=mesh,
        compiler_params=pltpu.CompilerParams(needs_layout_passes=False),
        out_type=[jax.ShapeDtypeStruct((NC * C, N), F32),
                  jax.ShapeDtypeStruct((NC * 2, N), F32)],
        scratch_types=[
            pltpu.VMEM((CPW, N), F32),
            pltpu.VMEM((CPW, N), F32),
            pltpu.VMEM((1, N), F32),
            pltpu.VMEM((1, N), F32),
            pltpu.VMEM((CPW, N), F32),
            pltpu.VMEM((1, N), F32),
            pltpu.VMEM((EH,), jnp.int32),
            pltpu.VMEM((EH,), jnp.int32),
        ],
    )
    def sc_edge(u_hbm, v_hbm, s_hbm, src_hbm, tgt_hbm, num_out, den_out,
                u_v, v_v, p_v, q_v, num_v, den_v, src_v, tgt_v):
        sc = lax.axis_index("c")
        sub = lax.axis_index("s")
        head = sub // (NS // 2)
        r0 = pl.multiple_of(sub * CPW, CPW)
        e0 = pl.multiple_of(sc * EH, EH)

        pltpu.sync_copy(u_hbm.at[pl.ds(r0, CPW)], u_v)
        pltpu.sync_copy(v_hbm.at[pl.ds(r0, CPW)], v_v)
        pltpu.sync_copy(s_hbm.at[pl.ds(head, 1)], p_v)
        pltpu.sync_copy(s_hbm.at[pl.ds(2 + head, 1)], q_v)
        pltpu.sync_copy(src_hbm.at[pl.ds(e0, EH)], src_v)
        pltpu.sync_copy(tgt_hbm.at[pl.ds(e0, EH)], tgt_v)

        zf = jnp.zeros((16,), F32)
        zi = jnp.zeros((16,), jnp.int32)

        @plsc.parallel_loop(0, N // 16, 1, unroll=8)
        def zero_acc(j):
            off = pl.multiple_of(j * 16, 16)
            for c in range(CPW):
                num_v[c, pl.ds(off, 16)] = zf
            den_v[0, pl.ds(off, 16)] = zf

        @plsc.parallel_loop(0, EH // 16, 1, unroll=2)
        def edge_body(g):
            base = pl.multiple_of(g * 16, 16)
            s16 = src_v[pl.ds(base, 16)]
            t16 = tgt_v[pl.ds(base, 16)]
            ps = plsc.load_gather(p_v, [zi, s16])
            qt = plsc.load_gather(q_v, [zi, t16])
            w = jnp.exp(ps + qt)
            plsc.addupdate_scatter(den_v, [zi, t16], w)
            for c in range(CPW):
                cv = jnp.full((16,), c, jnp.int32)
                us = plsc.load_gather(u_v, [cv, s16])
                vt = plsc.load_gather(v_v, [cv, t16])
                y = jnp.maximum(us + vt, 0.0)
                plsc.addupdate_scatter(num_v, [cv, t16], y * w)

        pltpu.sync_copy(num_v, num_out.at[pl.ds(sc * C + r0, CPW)])

        @pl.when(jnp.logical_or(sub == 0, sub == NS // 2))
        def _():
            pltpu.sync_copy(den_v, den_out.at[pl.ds(sc * 2 + head, 1)])

    return sc_edge


def _run_sc(c, u, v, s, src, tgt):
    return _make_sc_edge(c, c // 16)(u, v, s, src, tgt)


def _wargs(layer):
    h0, h1 = layer
    return (h0["f"]["W"], h0["f"]["b"], h1["f"]["W"], h1["f"]["b"],
            h0["w"]["W"], h1["w"]["W"])


def kernel(x, adj, src, tgt, Msrc, Mtgt, Mgraph, params):
    del adj, Msrc, Mtgt
    gat = params["gat"]
    dims = [(128, 32), (64, 64), (128, 64)]

    c1 = 2 * dims[0][1]
    u, v, s = _tc_call(functools.partial(_proj0_body, dims[0][0]),
                       [(c1, N), (c1, N), (8, N)], (x, *_wargs(gat[0])))
    num, den = _run_sc(c1, u, v, s, src, tgt)

    for li in (1, 2):
        cl = 2 * dims[li][1]
        u, v, s = _tc_call(functools.partial(_proj_mid_body, dims[li][0]),
                           [(cl, N), (cl, N), (8, N)],
                           (num, den, *_wargs(gat[li])))
        num, den = _run_sc(cl, u, v, s, src, tgt)

    (out,) = _tc_call(
        _final_body, [(G, 10)],
        (num, den, Mgraph,
         params["mlp"][0]["W"], params["mlp"][0]["b"],
         params["mlp"][1]["W"], params["mlp"][1]["b"]))
    return out


# speedup vs baseline: 1.4363x; 1.0404x over previous
import functools

import jax
import jax.numpy as jnp
from jax import lax
from jax.experimental import pallas as pl
from jax.experimental.pallas import tpu as pltpu
from jax.experimental.pallas import tpu_sc as plsc

N = 2048
E = 16384
G = 16
EPS = 1e-6
F32 = jnp.float32


def _dot(a, b, dims):
    return lax.dot_general(a, b, (dims, ((), ())), preferred_element_type=F32)


def _fold_s(S):
    m1 = jnp.max(S[0:1, :]) + jnp.max(S[2:3, :])
    m2 = jnp.max(S[1:2, :]) + jnp.max(S[3:4, :])
    return jnp.concatenate(
        [S[0:1] - m1, S[1:2] - m2, S[2:4], jnp.zeros((4, N), F32)], axis=0)


def _split_weights(d_in, w0f_ref, b0_ref, w1f_ref, b1_ref, w0w_ref, w1w_ref):
    w0, w1 = w0f_ref[...], w1f_ref[...]
    wu = jnp.concatenate([w0[:, :d_in], w1[:, :d_in]], axis=0)
    wv = jnp.concatenate([w0[:, d_in:], w1[:, d_in:]], axis=0)
    bu = jnp.concatenate([b0_ref[...], b1_ref[...]], axis=0)
    a0, a1 = w0w_ref[...], w1w_ref[...]
    ws = jnp.concatenate(
        [a0[:, :d_in], a1[:, :d_in], a0[:, d_in:], a1[:, d_in:]], axis=0)
    return wu, wv, bu, ws


def _proj0_body(d_in, x_ref, w0f_ref, b0_ref, w1f_ref, b1_ref, w0w_ref,
                w1w_ref, u_out, v_out, s_out):
    wu, wv, bu, ws = _split_weights(
        d_in, w0f_ref, b0_ref, w1f_ref, b1_ref, w0w_ref, w1w_ref)
    x = x_ref[...]
    u_out[...] = _dot(wu, x, ((1,), (1,))) + bu[:, None]
    v_out[...] = _dot(wv, x, ((1,), (1,)))
    s_out[...] = _fold_s(_dot(ws, x, ((1,), (1,))))


def _proj_mid_body(d_in, num_ref, den_ref, w0f_ref, b0_ref, w1f_ref, b1_ref,
                   w0w_ref, w1w_ref, u_out, v_out, s_out):
    wu, wv, bu, ws = _split_weights(
        d_in, w0f_ref, b0_ref, w1f_ref, b1_ref, w0w_ref, w1w_ref)
    c_in = num_ref.shape[0] // 2
    num = num_ref[:c_in] + num_ref[c_in:]
    den = den_ref[:2] + den_ref[2:]
    half = c_in // 2
    hc = jnp.concatenate([num[:half] / (den[0:1] + EPS),
                          num[half:] / (den[1:2] + EPS)], axis=0)
    u_out[...] = _dot(wu, hc, ((1,), (0,))) + bu[:, None]
    v_out[...] = _dot(wv, hc, ((1,), (0,)))
    s_out[...] = _fold_s(_dot(ws, hc, ((1,), (0,))))


def _final_body(num_ref, den_ref, mg_ref, w1_ref, b1_ref, w2_ref, b2_ref,
                out_ref):
    c_in = num_ref.shape[0] // 2
    num = num_ref[:c_in] + num_ref[c_in:]
    den = den_ref[:2] + den_ref[2:]
    half = c_in // 2
    hc = jnp.concatenate([num[:half] / (den[0:1] + EPS),
                          num[half:] / (den[1:2] + EPS)], axis=0)
    pooled = _dot(mg_ref[...], hc, ((0,), (1,)))
    z1 = jax.nn.relu(_dot(pooled, w1_ref[...], ((1,), (1,)))
                     + b1_ref[...][None, :])
    out_ref[...] = _dot(z1, w2_ref[...], ((1,), (1,))) + b2_ref[...][None, :]


def _tc_call(body, out_shapes, args):
    return pl.pallas_call(
        body,
        out_shape=[jax.ShapeDtypeStruct(s, F32) for s in out_shapes],
    )(*args)


@functools.cache
def _make_sc_edge(C, CPW):
    info = plsc.get_sparse_core_info()
    NC, NS = info.num_cores, info.num_subcores
    assert C == CPW * NS
    EH = E // NC
    mesh = plsc.VectorSubcoreMesh(core_axis_name="c", subcore_axis_name="s")

    @functools.partial(
        pl.kernel, mesh=mesh,
        compiler_params=pltpu.CompilerParams(needs_layout_passes=False),
        out_type=[jax.ShapeDtypeStruct((NC * C, N), F32),
                  jax.ShapeDtypeStruct((NC * 2, N), F32)],
        scratch_types=[
            pltpu.VMEM((CPW, N), F32),
            pltpu.VMEM((CPW, N), F32),
            pltpu.VMEM((1, N), F32),
            pltpu.VMEM((1, N), F32),
            pltpu.VMEM((CPW, N), F32),
            pltpu.VMEM((1, N), F32),
            pltpu.VMEM((EH,), jnp.int32),
            pltpu.VMEM((EH,), jnp.int32),
        ],
    )
    def sc_edge(u_hbm, v_hbm, s_hbm, src_hbm, tgt_hbm, num_out, den_out,
                u_v, v_v, p_v, q_v, num_v, den_v, src_v, tgt_v):
        sc = lax.axis_index("c")
        sub = lax.axis_index("s")
        head = sub // (NS // 2)
        r0 = pl.multiple_of(sub * CPW, CPW)
        e0 = pl.multiple_of(sc * EH, EH)

        pltpu.sync_copy(u_hbm.at[pl.ds(r0, CPW)], u_v)
        pltpu.sync_copy(v_hbm.at[pl.ds(r0, CPW)], v_v)
        pltpu.sync_copy(s_hbm.at[pl.ds(head, 1)], p_v)
        pltpu.sync_copy(s_hbm.at[pl.ds(2 + head, 1)], q_v)
        pltpu.sync_copy(src_hbm.at[pl.ds(e0, EH)], src_v)
        pltpu.sync_copy(tgt_hbm.at[pl.ds(e0, EH)], tgt_v)

        zf = jnp.zeros((16,), F32)
        zi = jnp.zeros((16,), jnp.int32)

        @plsc.parallel_loop(0, N // 16, 1, unroll=8)
        def zero_acc(j):
            off = pl.multiple_of(j * 16, 16)
            for c in range(CPW):
                num_v[c, pl.ds(off, 16)] = zf
            den_v[0, pl.ds(off, 16)] = zf

        @plsc.parallel_loop(0, EH // 16, 1, unroll=1)
        def edge_body(g):
            base = pl.multiple_of(g * 16, 16)
            s16 = src_v[pl.ds(base, 16)]
            t16 = tgt_v[pl.ds(base, 16)]
            ps = plsc.load_gather(p_v, [zi, s16])
            qt = plsc.load_gather(q_v, [zi, t16])
            w = jnp.exp(ps + qt)
            plsc.addupdate_scatter(den_v, [zi, t16], w)
            for c in range(CPW):
                cv = jnp.full((16,), c, jnp.int32)
                us = plsc.load_gather(u_v, [cv, s16])
                vt = plsc.load_gather(v_v, [cv, t16])
                y = jnp.maximum(us + vt, 0.0)
                plsc.addupdate_scatter(num_v, [cv, t16], y * w)

        pltpu.sync_copy(num_v, num_out.at[pl.ds(sc * C + r0, CPW)])

        @pl.when(jnp.logical_or(sub == 0, sub == NS // 2))
        def _():
            pltpu.sync_copy(den_v, den_out.at[pl.ds(sc * 2 + head, 1)])

    return sc_edge


def _run_sc(c, u, v, s, src, tgt):
    return _make_sc_edge(c, c // 16)(u, v, s, src, tgt)


def _wargs(layer):
    h0, h1 = layer
    return (h0["f"]["W"], h0["f"]["b"], h1["f"]["W"], h1["f"]["b"],
            h0["w"]["W"], h1["w"]["W"])


def kernel(x, adj, src, tgt, Msrc, Mtgt, Mgraph, params):
    del adj, Msrc, Mtgt
    gat = params["gat"]
    dims = [(128, 32), (64, 64), (128, 64)]

    c1 = 2 * dims[0][1]
    u, v, s = _tc_call(functools.partial(_proj0_body, dims[0][0]),
                       [(c1, N), (c1, N), (8, N)], (x, *_wargs(gat[0])))
    num, den = _run_sc(c1, u, v, s, src, tgt)

    for li in (1, 2):
        cl = 2 * dims[li][1]
        u, v, s = _tc_call(functools.partial(_proj_mid_body, dims[li][0]),
                           [(cl, N), (cl, N), (8, N)],
                           (num, den, *_wargs(gat[li])))
        num, den = _run_sc(cl, u, v, s, src, tgt)

    (out,) = _tc_call(
        _final_body, [(G, 10)],
        (num, den, Mgraph,
         params["mlp"][0]["W"], params["mlp"][0]["b"],
         params["mlp"][1]["W"], params["mlp"][1]["b"]))
    return out


# speedup vs baseline: 1.5568x; 1.0839x over previous
import functools

import jax
import jax.numpy as jnp
from jax import lax
from jax.experimental import pallas as pl
from jax.experimental.pallas import tpu as pltpu
from jax.experimental.pallas import tpu_sc as plsc

N = 2048
E = 16384
G = 16
EPS = 1e-6
F32 = jnp.float32


def _dot(a, b, dims):
    return lax.dot_general(a, b, (dims, ((), ())), preferred_element_type=F32)


def _fold_s(S):
    m1 = jnp.max(S[0:1, :]) + jnp.max(S[2:3, :])
    m2 = jnp.max(S[1:2, :]) + jnp.max(S[3:4, :])
    return jnp.concatenate(
        [S[0:1] - m1, S[1:2] - m2, S[2:4], jnp.zeros((4, N), F32)], axis=0)


def _split_weights(d_in, w0f_ref, b0_ref, w1f_ref, b1_ref, w0w_ref, w1w_ref):
    w0, w1 = w0f_ref[...], w1f_ref[...]
    wu = jnp.concatenate([w0[:, :d_in], w1[:, :d_in]], axis=0)
    wv = jnp.concatenate([w0[:, d_in:], w1[:, d_in:]], axis=0)
    bu = jnp.concatenate([b0_ref[...], b1_ref[...]], axis=0)
    a0, a1 = w0w_ref[...], w1w_ref[...]
    ws = jnp.concatenate(
        [a0[:, :d_in], a1[:, :d_in], a0[:, d_in:], a1[:, d_in:]], axis=0)
    return wu, wv, bu, ws


def _proj0_body(d_in, x_ref, w0f_ref, b0_ref, w1f_ref, b1_ref, w0w_ref,
                w1w_ref, u_out, v_out, s_out):
    wu, wv, bu, ws = _split_weights(
        d_in, w0f_ref, b0_ref, w1f_ref, b1_ref, w0w_ref, w1w_ref)
    x = x_ref[...]
    u_out[...] = _dot(wu, x, ((1,), (1,))) + bu[:, None]
    v_out[...] = _dot(wv, x, ((1,), (1,)))
    s_out[...] = _fold_s(_dot(ws, x, ((1,), (1,))))


def _proj_mid_body(d_in, num_ref, den_ref, w0f_ref, b0_ref, w1f_ref, b1_ref,
                   w0w_ref, w1w_ref, u_out, v_out, s_out):
    wu, wv, bu, ws = _split_weights(
        d_in, w0f_ref, b0_ref, w1f_ref, b1_ref, w0w_ref, w1w_ref)
    c_in = num_ref.shape[0] // 2
    num = num_ref[:c_in] + num_ref[c_in:]
    den = den_ref[:2] + den_ref[2:]
    half = c_in // 2
    hc = jnp.concatenate([num[:half] / (den[0:1] + EPS),
                          num[half:] / (den[1:2] + EPS)], axis=0)
    u_out[...] = _dot(wu, hc, ((1,), (0,))) + bu[:, None]
    v_out[...] = _dot(wv, hc, ((1,), (0,)))
    s_out[...] = _fold_s(_dot(ws, hc, ((1,), (0,))))


def _final_body(num_ref, den_ref, mg_ref, w1_ref, b1_ref, w2_ref, b2_ref,
                out_ref):
    c_in = num_ref.shape[0] // 2
    num = num_ref[:c_in] + num_ref[c_in:]
    den = den_ref[:2] + den_ref[2:]
    half = c_in // 2
    hc = jnp.concatenate([num[:half] / (den[0:1] + EPS),
                          num[half:] / (den[1:2] + EPS)], axis=0)
    pooled = _dot(mg_ref[...], hc, ((0,), (1,)))
    z1 = jax.nn.relu(_dot(pooled, w1_ref[...], ((1,), (1,)))
                     + b1_ref[...][None, :])
    out_ref[...] = _dot(z1, w2_ref[...], ((1,), (1,))) + b2_ref[...][None, :]


def _tc_call(body, out_shapes, args):
    return pl.pallas_call(
        body,
        out_shape=[jax.ShapeDtypeStruct(s, F32) for s in out_shapes],
    )(*args)


@functools.cache
def _make_sc_edge(C, CPW):
    info = plsc.get_sparse_core_info()
    NC, NS = info.num_cores, info.num_subcores
    assert C == CPW * NS
    EH = E // NC
    mesh = plsc.VectorSubcoreMesh(core_axis_name="c", subcore_axis_name="s")

    @functools.partial(
        pl.kernel, mesh=mesh,
        compiler_params=pltpu.CompilerParams(needs_layout_passes=False),
        out_type=[jax.ShapeDtypeStruct((NC * C, N), F32),
                  jax.ShapeDtypeStruct((NC * 2, N), F32)],
        scratch_types=[
            pltpu.VMEM((CPW, N), F32),
            pltpu.VMEM((CPW, N), F32),
            pltpu.VMEM((1, N), F32),
            pltpu.VMEM((1, N), F32),
            pltpu.VMEM((CPW, N), F32),
            pltpu.VMEM((1, N), F32),
            pltpu.VMEM((EH,), jnp.int32),
            pltpu.VMEM((EH,), jnp.int32),
            pltpu.SemaphoreType.DMA,
        ],
    )
    def sc_edge(u_hbm, v_hbm, s_hbm, src_hbm, tgt_hbm, num_out, den_out,
                u_v, v_v, p_v, q_v, num_v, den_v, src_v, tgt_v, sem):
        sc = lax.axis_index("c")
        sub = lax.axis_index("s")
        head = sub // (NS // 2)
        r0 = pl.multiple_of(sub * CPW, CPW)
        e0 = pl.multiple_of(sc * EH, EH)

        cps = [
            pltpu.async_copy(u_hbm.at[pl.ds(r0, CPW)], u_v, sem),
            pltpu.async_copy(v_hbm.at[pl.ds(r0, CPW)], v_v, sem),
            pltpu.async_copy(s_hbm.at[pl.ds(head, 1)], p_v, sem),
            pltpu.async_copy(s_hbm.at[pl.ds(2 + head, 1)], q_v, sem),
            pltpu.async_copy(src_hbm.at[pl.ds(e0, EH)], src_v, sem),
            pltpu.async_copy(tgt_hbm.at[pl.ds(e0, EH)], tgt_v, sem),
        ]

        zf = jnp.zeros((16,), F32)
        zi = jnp.zeros((16,), jnp.int32)

        @plsc.parallel_loop(0, N // 16, 1, unroll=8)
        def zero_acc(j):
            off = pl.multiple_of(j * 16, 16)
            for c in range(CPW):
                num_v[c, pl.ds(off, 16)] = zf
            den_v[0, pl.ds(off, 16)] = zf

        for cp in cps:
            cp.wait()

        @plsc.parallel_loop(0, EH // 16, 1, unroll=1)
        def edge_body(g):
            base = pl.multiple_of(g * 16, 16)
            s16 = src_v[pl.ds(base, 16)]
            t16 = tgt_v[pl.ds(base, 16)]
            ps = plsc.load_gather(p_v, [zi, s16])
            qt = plsc.load_gather(q_v, [zi, t16])
            w = jnp.exp(ps + qt)
            plsc.addupdate_scatter(den_v, [zi, t16], w)
            for c in range(CPW):
                cv = jnp.full((16,), c, jnp.int32)
                us = plsc.load_gather(u_v, [cv, s16])
                vt = plsc.load_gather(v_v, [cv, t16])
                y = jnp.maximum(us + vt, 0.0)
                plsc.addupdate_scatter(num_v, [cv, t16], y * w)

        pltpu.sync_copy(num_v, num_out.at[pl.ds(sc * C + r0, CPW)])

        @pl.when(jnp.logical_or(sub == 0, sub == NS // 2))
        def _():
            pltpu.sync_copy(den_v, den_out.at[pl.ds(sc * 2 + head, 1)])

    return sc_edge


def _run_sc(c, u, v, s, src, tgt):
    return _make_sc_edge(c, c // 16)(u, v, s, src, tgt)


def _wargs(layer):
    h0, h1 = layer
    return (h0["f"]["W"], h0["f"]["b"], h1["f"]["W"], h1["f"]["b"],
            h0["w"]["W"], h1["w"]["W"])


def kernel(x, adj, src, tgt, Msrc, Mtgt, Mgraph, params):
    del adj, Msrc, Mtgt
    gat = params["gat"]
    dims = [(128, 32), (64, 64), (128, 64)]

    c1 = 2 * dims[0][1]
    u, v, s = _tc_call(functools.partial(_proj0_body, dims[0][0]),
                       [(c1, N), (c1, N), (8, N)], (x, *_wargs(gat[0])))
    num, den = _run_sc(c1, u, v, s, src, tgt)

    for li in (1, 2):
        cl = 2 * dims[li][1]
        u, v, s = _tc_call(functools.partial(_proj_mid_body, dims[li][0]),
                           [(cl, N), (cl, N), (8, N)],
                           (num, den, *_wargs(gat[li])))
        num, den = _run_sc(cl, u, v, s, src, tgt)

    (out,) = _tc_call(
        _final_body, [(G, 10)],
        (num, den, Mgraph,
         params["mlp"][0]["W"], params["mlp"][0]["b"],
         params["mlp"][1]["W"], params["mlp"][1]["b"]))
    return out
